# baseline (speedup 1.0000x reference)
import functools

import jax
import jax.numpy as jnp
from jax import lax
from jax.experimental import pallas as pl
from jax.experimental.pallas import tpu as pltpu
from jax.experimental.pallas import tpu_sc as plsc

N_NODES = 100000
N_EDGES = 6400000
N_SLABS = 2

NC = 2
NS = 16
L = 16
NW = NC * NS

E_W = N_EDGES // N_SLABS // NW
CHUNK = 2000
N_CHUNKS = E_W // CHUNK
GROUPS = CHUNK // L
UNROLL = 5


@functools.partial(
    pl.kernel,
    out_type=jax.ShapeDtypeStruct((NW, N_NODES), jnp.float32),
    mesh=plsc.VectorSubcoreMesh(core_axis_name="c", subcore_axis_name="s"),
    compiler_params=pltpu.CompilerParams(needs_layout_passes=False),
    scratch_types=[
        pltpu.VMEM((CHUNK,), jnp.int32),
        pltpu.VMEM((CHUNK,), jnp.int32),
        pltpu.VMEM((CHUNK,), jnp.int32),
        pltpu.VMEM((CHUNK,), jnp.int32),
        pltpu.VMEM((CHUNK,), jnp.float32),
        pltpu.VMEM((CHUNK,), jnp.float32),
        pltpu.VMEM((N_NODES,), jnp.float32),
        pltpu.SemaphoreType.DMA,
        pltpu.SemaphoreType.DMA,
    ],
)
def _scatter_kernel(src_hbm, dst_hbm, val_hbm, out_hbm, sbuf0, sbuf1,
                    dbuf0, dbuf1, vbuf0, vbuf1, acc, sem0, sem1):
    wid = lax.axis_index("s") * NC + lax.axis_index("c")
    sems = (sem0, sem1)
    sbufs = (sbuf0, sbuf1)
    dbufs = (dbuf0, dbuf1)
    vbufs = (vbuf0, vbuf1)

    def zero_body(i, carry):
        acc[pl.ds(i * L, L)] = jnp.zeros((L,), jnp.float32)
        return carry

    lax.fori_loop(0, N_NODES // L, zero_body, 0)

    base = wid * E_W

    def start_fetch(c, slot):
        off = base + c * CHUNK
        pltpu.async_copy(src_hbm.at[pl.ds(off, CHUNK)], sbufs[slot], sems[slot])
        pltpu.async_copy(dst_hbm.at[pl.ds(off, CHUNK)], dbufs[slot], sems[slot])
        pltpu.async_copy(val_hbm.at[pl.ds(off, CHUNK)], vbufs[slot], sems[slot])

    def wait_fetch(c, slot):
        off = base + c * CHUNK
        pltpu.make_async_copy(src_hbm.at[pl.ds(off, CHUNK)], sbufs[slot], sems[slot]).wait()
        pltpu.make_async_copy(dst_hbm.at[pl.ds(off, CHUNK)], dbufs[slot], sems[slot]).wait()
        pltpu.make_async_copy(val_hbm.at[pl.ds(off, CHUNK)], vbufs[slot], sems[slot]).wait()

    def process(slot):
        def group_body(i, inner):
            for u in range(UNROLL):
                sl = pl.ds((i * UNROLL + u) * L, L)
                s = sbufs[slot][sl]
                d = dbufs[slot][sl]
                v = vbufs[slot][sl]
                plsc.addupdate_scatter(acc, [s], v)
                plsc.addupdate_scatter(acc, [d], -v)
            return inner

        lax.fori_loop(0, GROUPS // UNROLL, group_body, 0)

    start_fetch(0, 0)

    def ring_body(c2, carry):
        c = c2 * 2
        start_fetch(c + 1, 1)
        wait_fetch(c, 0)
        process(0)

        @pl.when(c + 2 < N_CHUNKS)
        def _():
            start_fetch(c + 2, 0)

        wait_fetch(c + 1, 1)
        process(1)
        return carry

    lax.fori_loop(0, N_CHUNKS // 2, ring_body, 0)

    pltpu.sync_copy(acc, out_hbm.at[wid])


def kernel(flow):
    half = N_EDGES // N_SLABS
    arrs = []
    for h in range(N_SLABS):
        slab = flow[h * half:(h + 1) * half]
        src = slab[:, 0].astype(jnp.int32)
        dst = slab[:, 1].astype(jnp.int32)
        val = slab[:, 2]
        partials = _scatter_kernel(src, dst, val)
        arrs.extend(partials[i] for i in range(NW))
    while len(arrs) > 1:
        arrs = [arrs[i] + arrs[i + 1] for i in range(0, len(arrs), 2)]
    net = lax.optimization_barrier(arrs[0])
    return jnp.sum(net)


# speedup vs baseline: 1.2529x; 1.2529x over previous
import functools

import jax
import jax.numpy as jnp
from jax import lax
from jax.experimental import pallas as pl
from jax.experimental.pallas import tpu as pltpu
from jax.experimental.pallas import tpu_sc as plsc

N_NODES = 100000
N_EDGES = 6400000
N_SLABS = 1

NC = 2
NS = 16
L = 16
NW = NC * NS

E_W = N_EDGES // N_SLABS // NW
CHUNK = 2000
N_CHUNKS = E_W // CHUNK
GROUPS = CHUNK // L
UNROLL = 5


@functools.partial(
    pl.kernel,
    out_type=jax.ShapeDtypeStruct((NW, N_NODES), jnp.float32),
    mesh=plsc.VectorSubcoreMesh(core_axis_name="c", subcore_axis_name="s"),
    compiler_params=pltpu.CompilerParams(needs_layout_passes=False),
    scratch_types=[
        pltpu.VMEM((CHUNK,), jnp.int32),
        pltpu.VMEM((CHUNK,), jnp.int32),
        pltpu.VMEM((CHUNK,), jnp.int32),
        pltpu.VMEM((CHUNK,), jnp.int32),
        pltpu.VMEM((CHUNK,), jnp.float32),
        pltpu.VMEM((CHUNK,), jnp.float32),
        pltpu.VMEM((N_NODES,), jnp.float32),
        pltpu.SemaphoreType.DMA,
        pltpu.SemaphoreType.DMA,
    ],
)
def _scatter_kernel(src_hbm, dst_hbm, val_hbm, out_hbm, sbuf0, sbuf1,
                    dbuf0, dbuf1, vbuf0, vbuf1, acc, sem0, sem1):
    wid = lax.axis_index("s") * NC + lax.axis_index("c")
    sems = (sem0, sem1)
    sbufs = (sbuf0, sbuf1)
    dbufs = (dbuf0, dbuf1)
    vbufs = (vbuf0, vbuf1)

    def zero_body(i, carry):
        acc[pl.ds(i * L, L)] = jnp.zeros((L,), jnp.float32)
        return carry

    lax.fori_loop(0, N_NODES // L, zero_body, 0)

    base = wid * E_W

    def start_fetch(c, slot):
        off = base + c * CHUNK
        pltpu.async_copy(src_hbm.at[pl.ds(off, CHUNK)], sbufs[slot], sems[slot])
        pltpu.async_copy(dst_hbm.at[pl.ds(off, CHUNK)], dbufs[slot], sems[slot])
        pltpu.async_copy(val_hbm.at[pl.ds(off, CHUNK)], vbufs[slot], sems[slot])

    def wait_fetch(c, slot):
        off = base + c * CHUNK
        pltpu.make_async_copy(src_hbm.at[pl.ds(off, CHUNK)], sbufs[slot], sems[slot]).wait()
        pltpu.make_async_copy(dst_hbm.at[pl.ds(off, CHUNK)], dbufs[slot], sems[slot]).wait()
        pltpu.make_async_copy(val_hbm.at[pl.ds(off, CHUNK)], vbufs[slot], sems[slot]).wait()

    def process(slot):
        def group_body(i, inner):
            for u in range(UNROLL):
                sl = pl.ds((i * UNROLL + u) * L, L)
                s = sbufs[slot][sl]
                d = dbufs[slot][sl]
                v = vbufs[slot][sl]
                plsc.addupdate_scatter(acc, [s], v)
                plsc.addupdate_scatter(acc, [d], -v)
            return inner

        lax.fori_loop(0, GROUPS // UNROLL, group_body, 0)

    start_fetch(0, 0)

    def ring_body(c2, carry):
        c = c2 * 2
        start_fetch(c + 1, 1)
        wait_fetch(c, 0)
        process(0)

        @pl.when(c + 2 < N_CHUNKS)
        def _():
            start_fetch(c + 2, 0)

        wait_fetch(c + 1, 1)
        process(1)
        return carry

    lax.fori_loop(0, N_CHUNKS // 2, ring_body, 0)

    pltpu.sync_copy(acc, out_hbm.at[wid])


def kernel(flow):
    half = N_EDGES // N_SLABS
    arrs = []
    for h in range(N_SLABS):
        slab = flow[h * half:(h + 1) * half]
        src = slab[:, 0].astype(jnp.int32)
        dst = slab[:, 1].astype(jnp.int32)
        val = slab[:, 2]
        partials = _scatter_kernel(src, dst, val)
        arrs.extend(partials[i] for i in range(NW))
    while len(arrs) > 1:
        arrs = [arrs[i] + arrs[i + 1] for i in range(0, len(arrs), 2)]
    net = lax.optimization_barrier(arrs[0])
    return jnp.sum(net)


# speedup vs baseline: 1.6766x; 1.3381x over previous
import functools

import jax
import jax.numpy as jnp
from jax import lax
from jax.experimental import pallas as pl
from jax.experimental.pallas import tpu as pltpu
from jax.experimental.pallas import tpu_sc as plsc

N_NODES = 100000
N_EDGES = 6400000
N_SLABS = 1

NC = 2
NS = 16
L = 16
NW = NC * NS

E_W = N_EDGES // N_SLABS // NW
CHUNK = 2000
N_CHUNKS = E_W // CHUNK
GROUPS = CHUNK // L
UNROLL = 5


@functools.partial(
    pl.kernel,
    out_type=jax.ShapeDtypeStruct((NW, N_NODES), jnp.float32),
    mesh=plsc.VectorSubcoreMesh(core_axis_name="c", subcore_axis_name="s"),
    compiler_params=pltpu.CompilerParams(needs_layout_passes=False),
    scratch_types=[
        pltpu.VMEM((CHUNK,), jnp.int32),
        pltpu.VMEM((CHUNK,), jnp.int32),
        pltpu.VMEM((CHUNK,), jnp.int32),
        pltpu.VMEM((CHUNK,), jnp.int32),
        pltpu.VMEM((CHUNK,), jnp.float32),
        pltpu.VMEM((CHUNK,), jnp.float32),
        pltpu.VMEM((N_NODES,), jnp.float32),
        pltpu.SemaphoreType.DMA,
        pltpu.SemaphoreType.DMA,
    ],
)
def _scatter_kernel(src_hbm, dst_hbm, val_hbm, out_hbm, sbuf0, sbuf1,
                    dbuf0, dbuf1, vbuf0, vbuf1, acc, sem0, sem1):
    wid = lax.axis_index("s") * NC + lax.axis_index("c")
    sems = (sem0, sem1)
    sbufs = (sbuf0, sbuf1)
    dbufs = (dbuf0, dbuf1)
    vbufs = (vbuf0, vbuf1)

    @plsc.parallel_loop(0, N_NODES // L, unroll=4)
    def _zero(i):
        acc[pl.ds(i * L, L)] = jnp.zeros((L,), jnp.float32)

    base = wid * E_W

    def start_fetch(c, slot):
        off = base + c * CHUNK
        pltpu.async_copy(src_hbm.at[pl.ds(off, CHUNK)], sbufs[slot], sems[slot])
        pltpu.async_copy(dst_hbm.at[pl.ds(off, CHUNK)], dbufs[slot], sems[slot])
        pltpu.async_copy(val_hbm.at[pl.ds(off, CHUNK)], vbufs[slot], sems[slot])

    def wait_fetch(c, slot):
        off = base + c * CHUNK
        pltpu.make_async_copy(src_hbm.at[pl.ds(off, CHUNK)], sbufs[slot], sems[slot]).wait()
        pltpu.make_async_copy(dst_hbm.at[pl.ds(off, CHUNK)], dbufs[slot], sems[slot]).wait()
        pltpu.make_async_copy(val_hbm.at[pl.ds(off, CHUNK)], vbufs[slot], sems[slot]).wait()

    def process(slot):
        @plsc.parallel_loop(0, GROUPS, unroll=UNROLL)
        def _groups(i):
            sl = pl.ds(i * L, L)
            s = sbufs[slot][sl]
            d = dbufs[slot][sl]
            v = vbufs[slot][sl]
            plsc.addupdate_scatter(acc, [s], v)
            plsc.addupdate_scatter(acc, [d], -v)

    start_fetch(0, 0)

    def ring_body(c2, carry):
        c = c2 * 2
        start_fetch(c + 1, 1)
        wait_fetch(c, 0)
        process(0)

        @pl.when(c + 2 < N_CHUNKS)
        def _():
            start_fetch(c + 2, 0)

        wait_fetch(c + 1, 1)
        process(1)
        return carry

    lax.fori_loop(0, N_CHUNKS // 2, ring_body, 0)

    pltpu.sync_copy(acc, out_hbm.at[wid])


def kernel(flow):
    half = N_EDGES // N_SLABS
    arrs = []
    for h in range(N_SLABS):
        slab = flow[h * half:(h + 1) * half]
        src = slab[:, 0].astype(jnp.int32)
        dst = slab[:, 1].astype(jnp.int32)
        val = slab[:, 2]
        partials = _scatter_kernel(src, dst, val)
        arrs.append(partials)
    stacked = arrs[0] if len(arrs) == 1 else jnp.concatenate(arrs, axis=0)
    net = lax.optimization_barrier(jnp.sum(stacked, axis=0))
    return jnp.sum(net)


# speedup vs baseline: 1.6767x; 1.0001x over previous
import functools

import jax
import jax.numpy as jnp
from jax import lax
from jax.experimental import pallas as pl
from jax.experimental.pallas import tpu as pltpu
from jax.experimental.pallas import tpu_sc as plsc

N_NODES = 100000
N_EDGES = 6400000
N_SLABS = 1

NC = 2
NS = 16
L = 16
NW = NC * NS

E_W = N_EDGES // N_SLABS // NW
CHUNK = 2000
N_CHUNKS = E_W // CHUNK
GROUPS = CHUNK // L
UNROLL = 10


@functools.partial(
    pl.kernel,
    out_type=jax.ShapeDtypeStruct((NW, N_NODES), jnp.float32),
    mesh=plsc.VectorSubcoreMesh(core_axis_name="c", subcore_axis_name="s"),
    compiler_params=pltpu.CompilerParams(needs_layout_passes=False),
    scratch_types=[
        pltpu.VMEM((CHUNK,), jnp.int32),
        pltpu.VMEM((CHUNK,), jnp.int32),
        pltpu.VMEM((CHUNK,), jnp.int32),
        pltpu.VMEM((CHUNK,), jnp.int32),
        pltpu.VMEM((CHUNK,), jnp.float32),
        pltpu.VMEM((CHUNK,), jnp.float32),
        pltpu.VMEM((N_NODES,), jnp.float32),
        pltpu.SemaphoreType.DMA,
        pltpu.SemaphoreType.DMA,
    ],
)
def _scatter_kernel(src_hbm, dst_hbm, val_hbm, out_hbm, sbuf0, sbuf1,
                    dbuf0, dbuf1, vbuf0, vbuf1, acc, sem0, sem1):
    wid = lax.axis_index("s") * NC + lax.axis_index("c")
    sems = (sem0, sem1)
    sbufs = (sbuf0, sbuf1)
    dbufs = (dbuf0, dbuf1)
    vbufs = (vbuf0, vbuf1)
    base = wid * E_W

    def start_fetch(c, slot):
        off = base + c * CHUNK
        pltpu.async_copy(src_hbm.at[pl.ds(off, CHUNK)], sbufs[slot], sems[slot])
        pltpu.async_copy(dst_hbm.at[pl.ds(off, CHUNK)], dbufs[slot], sems[slot])
        pltpu.async_copy(val_hbm.at[pl.ds(off, CHUNK)], vbufs[slot], sems[slot])

    def wait_fetch(c, slot):
        off = base + c * CHUNK
        pltpu.make_async_copy(src_hbm.at[pl.ds(off, CHUNK)], sbufs[slot], sems[slot]).wait()
        pltpu.make_async_copy(dst_hbm.at[pl.ds(off, CHUNK)], dbufs[slot], sems[slot]).wait()
        pltpu.make_async_copy(val_hbm.at[pl.ds(off, CHUNK)], vbufs[slot], sems[slot]).wait()

    def process(slot):
        @plsc.parallel_loop(0, GROUPS, unroll=UNROLL)
        def _groups(i):
            sl = pl.ds(i * L, L)
            s = sbufs[slot][sl]
            d = dbufs[slot][sl]
            v = vbufs[slot][sl]
            plsc.addupdate_scatter(acc, [s], v)
            plsc.addupdate_scatter(acc, [d], -v)

    start_fetch(0, 0)

    @plsc.parallel_loop(0, N_NODES // L, unroll=4)
    def _zero(i):
        acc[pl.ds(i * L, L)] = jnp.zeros((L,), jnp.float32)

    def ring_body(c2, carry):
        c = c2 * 2
        start_fetch(c + 1, 1)
        wait_fetch(c, 0)
        process(0)

        @pl.when(c + 2 < N_CHUNKS)
        def _():
            start_fetch(c + 2, 0)

        wait_fetch(c + 1, 1)
        process(1)
        return carry

    lax.fori_loop(0, N_CHUNKS // 2, ring_body, 0)

    pltpu.sync_copy(acc, out_hbm.at[wid])


def kernel(flow):
    half = N_EDGES // N_SLABS
    arrs = []
    for h in range(N_SLABS):
        slab = flow[h * half:(h + 1) * half]
        src = slab[:, 0].astype(jnp.int32)
        dst = slab[:, 1].astype(jnp.int32)
        val = slab[:, 2]
        partials = _scatter_kernel(src, dst, val)
        arrs.append(partials)
    stacked = arrs[0] if len(arrs) == 1 else jnp.concatenate(arrs, axis=0)
    net = lax.optimization_barrier(jnp.sum(stacked, axis=0))
    return jnp.sum(net)
